# baseline (speedup 1.0000x reference)
import jax
import jax.numpy as jnp
from jax import lax
from jax.experimental import pallas as pl
from jax.experimental.pallas import tpu as pltpu
from jax.experimental.pallas import tpu_sc as plsc

B = 16384
D = 64
V = 100000
NUM_CORES = 2
NUM_SUBCORES = 16
NW = NUM_CORES * NUM_SUBCORES
ROWS_PER_W = D // NW
IDX_CHUNK = 8192
LANES = 16
UNROLL = 4


def _gather_body(idx_hbm, table_hbm, out_hbm, idx_v, row_v, out_v, sem, sem_i):
    wid = lax.axis_index("s") * NUM_CORES + lax.axis_index("c")
    zeros16 = jnp.zeros((LANES,), jnp.int32)

    idx_cp = pltpu.async_copy(idx_hbm, idx_v, sem_i)
    for r in range(ROWS_PER_W):
        jj = wid * ROWS_PER_W + r
        pltpu.async_copy(table_hbm.at[pl.ds(jj, 1), :], row_v, sem).wait()
        if r == 0:
            idx_cp.wait()
        for chunk in range(B // IDX_CHUNK):

            def body(i, _, chunk=chunk):
                for u in range(UNROLL):
                    o = i * (LANES * UNROLL) + u * LANES
                    iv = idx_v[pl.ds(chunk * IDX_CHUNK + o, LANES)]
                    vals = plsc.load_gather(row_v, [zeros16, iv])
                    out_v[0, pl.ds(o, LANES)] = vals
                return ()

            lax.fori_loop(0, IDX_CHUNK // (LANES * UNROLL), body, ())
            pltpu.sync_copy(
                out_v,
                out_hbm.at[pl.ds(jj, 1), pl.ds(chunk * IDX_CHUNK, IDX_CHUNK)],
            )


_sc_gather = pl.kernel(
    _gather_body,
    mesh=plsc.VectorSubcoreMesh(core_axis_name="c", subcore_axis_name="s"),
    out_type=jax.ShapeDtypeStruct((D, B), jnp.float32),
    scratch_types=[
        pltpu.VMEM((B,), jnp.int32),
        pltpu.VMEM((1, V), jnp.float32),
        pltpu.VMEM((1, IDX_CHUNK), jnp.float32),
        pltpu.SemaphoreType.DMA,
        pltpu.SemaphoreType.DMA,
    ],
    compiler_params=pltpu.CompilerParams(needs_layout_passes=False),
)


@jax.jit
def kernel(input, action_embedding):
    idx = input.reshape(B)
    out_t = _sc_gather(idx, action_embedding.T)
    return out_t.T


# speedup vs baseline: 1.1167x; 1.1167x over previous
import jax
import jax.numpy as jnp
from jax import lax
from jax.experimental import pallas as pl
from jax.experimental.pallas import tpu as pltpu
from jax.experimental.pallas import tpu_sc as plsc

B = 16384
D = 64
V = 100000
NUM_CORES = 2
NUM_SUBCORES = 16
NW = NUM_CORES * NUM_SUBCORES
ROWS_PER_W = D // NW
IDX_CHUNK = 8192
LANES = 16
UNROLL = 8


def _gather_body(idx_hbm, table_hbm, out_hbm, idx_v, row_v, out_v, sem, sem_i):
    wid = lax.axis_index("s") * NUM_CORES + lax.axis_index("c")
    zeros16 = jnp.zeros((LANES,), jnp.int32)

    idx_cp = pltpu.async_copy(idx_hbm, idx_v, sem_i)
    for r in range(ROWS_PER_W):
        jj = wid * ROWS_PER_W + r
        pltpu.async_copy(table_hbm.at[pl.ds(jj, 1), :], row_v, sem).wait()
        if r == 0:
            idx_cp.wait()
        for chunk in range(B // IDX_CHUNK):

            def body(i, _, chunk=chunk):
                for u in range(UNROLL):
                    o = i * (LANES * UNROLL) + u * LANES
                    iv = idx_v[pl.ds(chunk * IDX_CHUNK + o, LANES)]
                    vals = plsc.load_gather(row_v, [zeros16, iv])
                    out_v[0, pl.ds(o, LANES)] = vals
                return ()

            lax.fori_loop(0, IDX_CHUNK // (LANES * UNROLL), body, ())
            pltpu.sync_copy(
                out_v,
                out_hbm.at[pl.ds(jj, 1), pl.ds(chunk * IDX_CHUNK, IDX_CHUNK)],
            )


_sc_gather = pl.kernel(
    _gather_body,
    mesh=plsc.VectorSubcoreMesh(core_axis_name="c", subcore_axis_name="s"),
    out_type=jax.ShapeDtypeStruct((D, B), jnp.float32),
    scratch_types=[
        pltpu.VMEM((B,), jnp.int32),
        pltpu.VMEM((1, V), jnp.float32),
        pltpu.VMEM((1, IDX_CHUNK), jnp.float32),
        pltpu.SemaphoreType.DMA,
        pltpu.SemaphoreType.DMA,
    ],
    compiler_params=pltpu.CompilerParams(needs_layout_passes=False),
)


@jax.jit
def kernel(input, action_embedding):
    idx = input.reshape(B)
    out_t = _sc_gather(idx, action_embedding.T)
    return out_t.T


# speedup vs baseline: 1.2970x; 1.1614x over previous
import jax
import jax.numpy as jnp
from jax import lax
from jax.experimental import pallas as pl
from jax.experimental.pallas import tpu as pltpu
from jax.experimental.pallas import tpu_sc as plsc

B = 16384
D = 64
V = 100000
NUM_CORES = 2
NUM_SUBCORES = 16
NW = NUM_CORES * NUM_SUBCORES
ROWS_PER_W = D // NW
IDX_CHUNK = 8192
LANES = 16
UNROLL = 8


def _gather_body(idx_hbm, table_hbm, out_hbm, idx_v, row_v, out_v, sem, sem_i):
    wid = lax.axis_index("s") * NUM_CORES + lax.axis_index("c")
    zeros16 = jnp.zeros((LANES,), jnp.int32)

    idx_cp = pltpu.async_copy(idx_hbm, idx_v, sem_i)
    for r in range(ROWS_PER_W):
        jj = wid * ROWS_PER_W + r
        pltpu.async_copy(table_hbm.at[pl.ds(jj, 1), :], row_v, sem).wait()
        if r == 0:
            idx_cp.wait()
        for chunk in range(B // IDX_CHUNK):

            @plsc.parallel_loop(0, IDX_CHUNK // LANES, unroll=UNROLL)
            def body(i, chunk=chunk):
                o = i * LANES
                iv = idx_v[pl.ds(chunk * IDX_CHUNK + o, LANES)]
                vals = plsc.load_gather(row_v, [zeros16, iv])
                out_v[0, pl.ds(o, LANES)] = vals
            pltpu.sync_copy(
                out_v,
                out_hbm.at[pl.ds(jj, 1), pl.ds(chunk * IDX_CHUNK, IDX_CHUNK)],
            )


_sc_gather = pl.kernel(
    _gather_body,
    mesh=plsc.VectorSubcoreMesh(core_axis_name="c", subcore_axis_name="s"),
    out_type=jax.ShapeDtypeStruct((D, B), jnp.float32),
    scratch_types=[
        pltpu.VMEM((B,), jnp.int32),
        pltpu.VMEM((1, V), jnp.float32),
        pltpu.VMEM((1, IDX_CHUNK), jnp.float32),
        pltpu.SemaphoreType.DMA,
        pltpu.SemaphoreType.DMA,
    ],
    compiler_params=pltpu.CompilerParams(needs_layout_passes=False),
)


@jax.jit
def kernel(input, action_embedding):
    idx = input.reshape(B)
    out_t = _sc_gather(idx, action_embedding.T)
    return out_t.T


# speedup vs baseline: 1.2999x; 1.0022x over previous
import jax
import jax.numpy as jnp
from jax import lax
from jax.experimental import pallas as pl
from jax.experimental.pallas import tpu as pltpu
from jax.experimental.pallas import tpu_sc as plsc

B = 16384
D = 64
V = 100000
NUM_CORES = 2
NUM_SUBCORES = 16
NW = NUM_CORES * NUM_SUBCORES
ROWS_PER_W = D // NW
IDX_CHUNK = 8192
LANES = 16
UNROLL = 16


def _gather_body(idx_hbm, table_hbm, out_hbm, idx_v, row_v, out_v, sem, sem_i):
    wid = lax.axis_index("s") * NUM_CORES + lax.axis_index("c")
    zeros16 = jnp.zeros((LANES,), jnp.int32)

    idx_cp = pltpu.async_copy(idx_hbm, idx_v, sem_i)
    for r in range(ROWS_PER_W):
        jj = wid * ROWS_PER_W + r
        pltpu.async_copy(table_hbm.at[pl.ds(jj, 1), :], row_v, sem).wait()
        if r == 0:
            idx_cp.wait()
        for chunk in range(B // IDX_CHUNK):

            @plsc.parallel_loop(0, IDX_CHUNK // LANES, unroll=UNROLL)
            def body(i, chunk=chunk):
                o = i * LANES
                iv = idx_v[pl.ds(chunk * IDX_CHUNK + o, LANES)]
                vals = plsc.load_gather(row_v, [zeros16, iv])
                out_v[0, pl.ds(o, LANES)] = vals
            pltpu.sync_copy(
                out_v,
                out_hbm.at[pl.ds(jj, 1), pl.ds(chunk * IDX_CHUNK, IDX_CHUNK)],
            )


_sc_gather = pl.kernel(
    _gather_body,
    mesh=plsc.VectorSubcoreMesh(core_axis_name="c", subcore_axis_name="s"),
    out_type=jax.ShapeDtypeStruct((D, B), jnp.float32),
    scratch_types=[
        pltpu.VMEM((B,), jnp.int32),
        pltpu.VMEM((1, V), jnp.float32),
        pltpu.VMEM((1, IDX_CHUNK), jnp.float32),
        pltpu.SemaphoreType.DMA,
        pltpu.SemaphoreType.DMA,
    ],
    compiler_params=pltpu.CompilerParams(needs_layout_passes=False),
)


@jax.jit
def kernel(input, action_embedding):
    idx = input.reshape(B)
    out_t = _sc_gather(idx, action_embedding.T)
    return out_t.T
